# baseline (speedup 1.0000x reference)
import jax
import jax.numpy as jnp
from jax import lax
from jax.experimental import pallas as pl
from jax.experimental.pallas import tpu as pltpu
from jax.experimental.pallas import tpu_sc as plsc

VOCAB = 1000000
EMBED_DIM = 64
BATCH = 16384
HIST = 50
NUM_IDX = BATCH * HIST

NUM_WORKERS = 32
B_PER_W = NUM_IDX // NUM_WORKERS
ROWS_PER_W = BATCH // NUM_WORKERS
CHUNKB = 4
CHUNK = CHUNKB * HIST
GATHER_SPLITS = ((0, 128), (128, 72))
NBUF = 4
N_CHUNKS = ROWS_PER_W // CHUNKB
N_GROUPS = N_CHUNKS // NBUF


def _sc_gather(x_flat, table):
    mesh = plsc.VectorSubcoreMesh(core_axis_name="c", subcore_axis_name="s")

    @pl.kernel(
        out_type=jax.ShapeDtypeStruct((BATCH, HIST, EMBED_DIM), jnp.float32),
        mesh=mesh,
        compiler_params=pltpu.CompilerParams(use_tc_tiling_on_sc=False),
        scratch_types=(
            [
                pltpu.VMEM((B_PER_W,), jnp.int32),
                pltpu.VMEM((NBUF, CHUNK, EMBED_DIM), jnp.float32),
            ]
            + [pltpu.SemaphoreType.DMA] * (2 * NBUF)
        ),
    )
    def gather_kernel(table_hbm, idx_hbm, out_hbm, idx_all, rows_v, *sems):
        gsem = sems[:NBUF]
        wsem = sems[NBUF:]
        wid = lax.axis_index("s") * 2 + lax.axis_index("c")
        base = wid * B_PER_W
        rbase = wid * ROWS_PER_W
        pltpu.sync_copy(idx_hbm.at[pl.ds(base, B_PER_W)], idx_all)

        def enq_gather(c, b):
            for off, n in GATHER_SPLITS:
                pltpu.async_copy(
                    table_hbm.at[idx_all.at[pl.ds(c * CHUNK + off, n)]],
                    rows_v.at[b, pl.ds(off, n)],
                    gsem[b],
                )

        def wait_gather(c, b):
            for off, n in GATHER_SPLITS:
                pltpu.make_async_copy(
                    table_hbm.at[idx_all.at[pl.ds(c * CHUNK + off, n)]],
                    rows_v.at[b, pl.ds(off, n)],
                    gsem[b],
                ).wait()

        def enq_write(c, b):
            for j in range(CHUNKB):
                pltpu.async_copy(
                    rows_v.at[b, pl.ds(j * HIST, HIST)],
                    out_hbm.at[rbase + c * CHUNKB + j],
                    wsem[b],
                )

        def wait_write(c, b):
            for j in range(CHUNKB):
                pltpu.make_async_copy(
                    rows_v.at[b, pl.ds(j * HIST, HIST)],
                    out_hbm.at[rbase + c * CHUNKB + j],
                    wsem[b],
                ).wait()

        for b in range(NBUF):
            enq_gather(b, b)
        for b in range(NBUF):
            wait_gather(b, b)
            enq_write(b, b)

        @pl.loop(1, N_GROUPS)
        def _(g):
            c0 = g * NBUF
            for b in range(NBUF):
                wait_write(c0 - NBUF + b, b)
                enq_gather(c0 + b, b)
            for b in range(NBUF):
                wait_gather(c0 + b, b)
                enq_write(c0 + b, b)

        for b in range(NBUF):
            wait_write(N_CHUNKS - NBUF + b, b)

    return gather_kernel(table, x_flat)


@jax.jit
def kernel(x, table):
    x_flat = x.reshape(NUM_IDX).astype(jnp.int32)
    return _sc_gather(x_flat, table)


# speedup vs baseline: 1.3193x; 1.3193x over previous
import jax
import jax.numpy as jnp
from jax import lax
from jax.experimental import pallas as pl
from jax.experimental.pallas import tpu as pltpu
from jax.experimental.pallas import tpu_sc as plsc

VOCAB = 1000000
EMBED_DIM = 64
BATCH = 16384
HIST = 50
NUM_IDX = BATCH * HIST

NUM_WORKERS = 32
B_PER_W = NUM_IDX // NUM_WORKERS
ROWS_PER_W = BATCH // NUM_WORKERS
CHUNKB = 4
CHUNK = CHUNKB * HIST
GATHER_SPLITS = ((0, 128), (128, 72))
NBUF = 4
N_CHUNKS = ROWS_PER_W // CHUNKB
N_GROUPS = N_CHUNKS // NBUF


HALF_VOCAB = VOCAB // 2
TW = 3200
TGRID = (VOCAB + 2 * TW - 1) // (2 * TW)


def _tc_pack(t_transposed):
    def body(x_ref, o_ref):
        o_ref[:, 0:EMBED_DIM] = x_ref[:, 0:TW].T
        o_ref[:, EMBED_DIM:] = x_ref[:, TW:].T

    return pl.pallas_call(
        body,
        grid=(TGRID,),
        in_specs=[pl.BlockSpec((EMBED_DIM, 2 * TW), lambda i: (0, i))],
        out_specs=pl.BlockSpec((TW, 2 * EMBED_DIM), lambda i: (i, 0)),
        out_shape=jax.ShapeDtypeStruct((HALF_VOCAB, 2 * EMBED_DIM), jnp.float32),
        compiler_params=pltpu.CompilerParams(
            dimension_semantics=("parallel",),
        ),
    )(t_transposed)


def _sc_gather(x_flat, table):
    mesh = plsc.VectorSubcoreMesh(core_axis_name="c", subcore_axis_name="s")

    @pl.kernel(
        out_type=jax.ShapeDtypeStruct((BATCH, HIST, EMBED_DIM), jnp.float32),
        mesh=mesh,
        compiler_params=pltpu.CompilerParams(use_tc_tiling_on_sc=False),
        scratch_types=(
            [
                pltpu.VMEM((B_PER_W,), jnp.int32),
                pltpu.VMEM((NBUF, CHUNK, EMBED_DIM), jnp.float32),
            ]
            + [pltpu.SemaphoreType.DMA] * (2 * NBUF)
        ),
    )
    def gather_kernel(table_hbm, idx_hbm, out_hbm, idx_all, rows_v, *sems):
        gsem = sems[:NBUF]
        wsem = sems[NBUF:]
        wid = lax.axis_index("s") * 2 + lax.axis_index("c")
        base = wid * B_PER_W
        rbase = wid * ROWS_PER_W
        pltpu.sync_copy(idx_hbm.at[pl.ds(base, B_PER_W)], idx_all)

        def enq_gather(c, b):
            for off, n in GATHER_SPLITS:
                pltpu.async_copy(
                    table_hbm.at[idx_all.at[pl.ds(c * CHUNK + off, n)]],
                    rows_v.at[b, pl.ds(off, n)],
                    gsem[b],
                )

        def wait_gather(c, b):
            for off, n in GATHER_SPLITS:
                pltpu.make_async_copy(
                    table_hbm.at[idx_all.at[pl.ds(c * CHUNK + off, n)]],
                    rows_v.at[b, pl.ds(off, n)],
                    gsem[b],
                ).wait()

        def enq_write(c, b):
            for j in range(CHUNKB):
                pltpu.async_copy(
                    rows_v.at[b, pl.ds(j * HIST, HIST)],
                    out_hbm.at[rbase + c * CHUNKB + j],
                    wsem[b],
                )

        def wait_write(c, b):
            for j in range(CHUNKB):
                pltpu.make_async_copy(
                    rows_v.at[b, pl.ds(j * HIST, HIST)],
                    out_hbm.at[rbase + c * CHUNKB + j],
                    wsem[b],
                ).wait()

        for b in range(NBUF):
            enq_gather(b, b)
        for b in range(NBUF):
            wait_gather(b, b)
            enq_write(b, b)

        @pl.loop(1, N_GROUPS)
        def _(g):
            c0 = g * NBUF
            for b in range(NBUF):
                wait_write(c0 - NBUF + b, b)
                enq_gather(c0 + b, b)
            for b in range(NBUF):
                wait_gather(c0 + b, b)
                enq_write(c0 + b, b)

        for b in range(NBUF):
            wait_write(N_CHUNKS - NBUF + b, b)

    return gather_kernel(table, x_flat)


@jax.jit
def kernel(x, table):
    x_flat = x.reshape(NUM_IDX).astype(jnp.int32)
    blk = x_flat // (2 * TW)
    j = x_flat - blk * (2 * TW)
    half = j // TW
    x_lin = blk * (2 * TW) + 2 * (j - half * TW) + half
    table_lin = _tc_pack(table.T).reshape(VOCAB, EMBED_DIM)
    return _sc_gather(x_lin, table_lin)
